# Initial kernel scaffold; baseline (speedup 1.0000x reference)
import jax
import jax.numpy as jnp
from jax.experimental import pallas as pl


def kernel(node_features, edge_features, edge_indices, hidden_state, W1, b1, W2, b2, gru_kernel, gru_rkernel, gru_bias):
    raise NotImplementedError("write your pallas kernel here")


# speedup vs baseline: 1.0905x; 1.0905x over previous
import functools

import jax
import jax.numpy as jnp
import numpy as np
from jax import lax
from jax.experimental import pallas as pl
from jax.experimental.pallas import tpu as pltpu
from jax.experimental.pallas import tpu_sc as plsc

U = 32
ED = 16
N_NODES = 10000
N_EDGES = 160000

CHUNK = 128
EP = 163840
NROWS = EP // CHUNK
NPAD = 10048
STRIPE = NPAD // 16
DUMMY = N_NODES

BE = 1024
HIGH = lax.Precision.HIGHEST


def _sc_mesh():
    return plsc.VectorSubcoreMesh(core_axis_name="core", subcore_axis_name="subcore")


SC_PARAMS = pltpu.CompilerParams(use_tc_tiling_on_sc=False)


def _gather(node_features, src_rows):
    @functools.partial(
        pl.kernel,
        out_type=jax.ShapeDtypeStruct((EP, U), jnp.float32),
        mesh=_sc_mesh(),
        compiler_params=SC_PARAMS,
    )
    def gather_kernel(table_hbm, idx_hbm, out_hbm):
        def body(i_vmem, o_vmem):
            pltpu.sync_copy(table_hbm.at[i_vmem.at[0]], o_vmem)

        pltpu.emit_pipeline(
            body,
            grid=(NROWS,),
            in_specs=[pl.BlockSpec((1, CHUNK), lambda i: (0, i))],
            out_specs=[pl.BlockSpec((CHUNK, U), lambda i: (i, 0))],
            core_axis_name=("core", "subcore"),
            dimension_semantics=(pltpu.PARALLEL,),
        )(idx_hbm, out_hbm)

    return gather_kernel(node_features, src_rows)


def _dense_body(ef_ref, src_ref, w1_ref, b1_ref, w2_ref, b2_ref, sel_ref, out_ref):
    h = jnp.maximum(
        jnp.dot(ef_ref[...], w1_ref[...], precision=HIGH) + b1_ref[...], 0.0
    )
    full = jnp.dot(h, w2_ref[...], precision=HIGH) + b2_ref[...]
    src = src_ref[...]
    src_t = jnp.concatenate([src] * U, axis=1)
    out_ref[...] = jnp.dot(full * src_t, sel_ref[...], precision=HIGH)


def _dense(ef_p, src_feat, W1, b1, W2, b2, sel):
    grid = EP // BE
    return pl.pallas_call(
        _dense_body,
        grid=(grid,),
        in_specs=[
            pl.BlockSpec((BE, ED), lambda i: (i, 0)),
            pl.BlockSpec((BE, U), lambda i: (i, 0)),
            pl.BlockSpec((ED, U), lambda i: (0, 0)),
            pl.BlockSpec((1, U), lambda i: (0, 0)),
            pl.BlockSpec((U, U * U), lambda i: (0, 0)),
            pl.BlockSpec((1, U * U), lambda i: (0, 0)),
            pl.BlockSpec((U * U, U), lambda i: (0, 0)),
        ],
        out_specs=pl.BlockSpec((BE, U), lambda i: (i, 0)),
        out_shape=jax.ShapeDtypeStruct((EP, U), jnp.float32),
    )(ef_p, src_feat, W1, b1, W2, b2, sel)


def _scatter(messages, dst_rows, zeros32, zeros16, ones16):
    @functools.partial(
        pl.kernel,
        out_type=[
            jax.ShapeDtypeStruct((2, NPAD, U), jnp.float32),
            jax.ShapeDtypeStruct((2, NPAD, 16), jnp.float32),
        ],
        mesh=_sc_mesh(),
        compiler_params=SC_PARAMS,
        scratch_types=[
            pltpu.VMEM_SHARED((NPAD, U), jnp.float32),
            pltpu.VMEM_SHARED((NPAD, 16), jnp.float32),
            pltpu.VMEM((CHUNK, 16), jnp.float32),
        ],
    )
    def scatter_kernel(msg_hbm, idx_hbm, z32_hbm, z16_hbm, ones_hbm,
                       sums_out, cnt_out, sums_sh, cnt_sh, ones_v):
        cid = lax.axis_index("core")
        sid = lax.axis_index("subcore")
        base = sid * STRIPE
        pltpu.sync_copy(z32_hbm, sums_sh.at[pl.ds(base, STRIPE)])
        pltpu.sync_copy(z16_hbm, cnt_sh.at[pl.ds(base, STRIPE)])
        pltpu.sync_copy(ones_hbm, ones_v)
        plsc.subcore_barrier()

        def body(m_vmem, i_vmem):
            pltpu.sync_copy(m_vmem, sums_sh.at[i_vmem.at[0]], add=True)
            pltpu.sync_copy(ones_v, cnt_sh.at[i_vmem.at[0]], add=True)

        pltpu.emit_pipeline(
            body,
            grid=(NROWS,),
            in_specs=[
                pl.BlockSpec((CHUNK, U), lambda i: (i, 0)),
                pl.BlockSpec((1, CHUNK), lambda i: (0, i)),
            ],
            out_specs=[],
            core_axis_name=("core", "subcore"),
            dimension_semantics=(pltpu.PARALLEL,),
        )(msg_hbm, idx_hbm)
        plsc.subcore_barrier()
        pltpu.sync_copy(sums_sh.at[pl.ds(base, STRIPE)],
                        sums_out.at[cid, pl.ds(base, STRIPE)])
        pltpu.sync_copy(cnt_sh.at[pl.ds(base, STRIPE)],
                        cnt_out.at[cid, pl.ds(base, STRIPE)])

    return scatter_kernel(messages, dst_rows, zeros32, zeros16, ones16)


def _gru_body(sums_ref, cnt_ref, gk_ref, grk_ref, gb_ref, h0_ref, out_ref, xm_ref):
    sums = sums_ref[0, 0:N_NODES, :] + sums_ref[1, 0:N_NODES, :]
    cnt = cnt_ref[0, 0:N_NODES, 0:1] + cnt_ref[1, 0:N_NODES, 0:1]
    agg = sums / jnp.maximum(cnt, 1.0)
    xm_ref[...] = jnp.dot(agg, gk_ref[...], precision=HIGH) + gb_ref[0:1, :]

    grk = grk_ref[...]
    gb1 = gb_ref[1:2, :]

    def step(t, h):
        xm = xm_ref[pl.ds(t, 1), :]
        hm = jnp.dot(h, grk, precision=HIGH) + gb1
        z = jax.nn.sigmoid(xm[:, 0:U] + hm[:, 0:U])
        r = jax.nn.sigmoid(xm[:, U:2 * U] + hm[:, U:2 * U])
        hh = jnp.tanh(xm[:, 2 * U:3 * U] + r * hm[:, 2 * U:3 * U])
        return z * h + (1.0 - z) * hh

    out_ref[...] = lax.fori_loop(0, N_NODES, step, h0_ref[...])


def _gru(sums_p, cnt_p, gru_kernel, gru_rkernel, gru_bias, hidden_state):
    return pl.pallas_call(
        _gru_body,
        grid=(1,),
        in_specs=[
            pl.BlockSpec((2, NPAD, U), lambda i: (0, 0, 0)),
            pl.BlockSpec((2, NPAD, 16), lambda i: (0, 0, 0)),
            pl.BlockSpec((U, 3 * U), lambda i: (0, 0)),
            pl.BlockSpec((U, 3 * U), lambda i: (0, 0)),
            pl.BlockSpec((2, 3 * U), lambda i: (0, 0)),
            pl.BlockSpec((1, U), lambda i: (0, 0)),
        ],
        out_specs=pl.BlockSpec((1, U), lambda i: (0, 0)),
        out_shape=jax.ShapeDtypeStruct((1, U), jnp.float32),
        scratch_shapes=[pltpu.VMEM((N_NODES, 3 * U), jnp.float32)],
    )(sums_p, cnt_p, gru_kernel, gru_rkernel, gru_bias, hidden_state)


def kernel(node_features, edge_features, edge_indices, hidden_state,
           W1, b1, W2, b2, gru_kernel, gru_rkernel, gru_bias):
    pad = EP - N_EDGES
    src_idx = jnp.concatenate(
        [edge_indices[0], jnp.zeros((pad,), jnp.int32)]).reshape(1, EP)
    dst_idx = jnp.concatenate(
        [edge_indices[1], jnp.full((pad,), DUMMY, jnp.int32)]).reshape(1, EP)
    ef_p = jnp.concatenate(
        [edge_features, jnp.zeros((pad, ED), jnp.float32)], axis=0)

    sel = jnp.asarray(np.repeat(np.eye(U, dtype=np.float32), U, axis=0))
    zeros32 = jnp.zeros((STRIPE, U), jnp.float32)
    zeros16 = jnp.zeros((STRIPE, 16), jnp.float32)
    ones16 = jnp.ones((CHUNK, 16), jnp.float32)

    src_feat = _gather(node_features, src_idx)
    messages = _dense(ef_p, src_feat, W1, b1.reshape(1, U), W2,
                      b2.reshape(1, U * U), sel)
    sums_p, cnt_p = _scatter(messages, dst_idx, zeros32, zeros16, ones16)
    new_state = _gru(sums_p, cnt_p, gru_kernel, gru_rkernel, gru_bias,
                     hidden_state)
    return new_state[0], new_state
